# baseline (device time: 115160 ns/iter reference)
import jax
import jax.numpy as jnp
from jax import lax
from jax.experimental import pallas as pl
from jax.experimental.pallas import tpu as pltpu

N_DEV = 4
SQ = 512
SKV = 2048
HQ = 8
DH = 128
DM = HQ * DH
SCALE = 0.08838834764831843
LOG2E = 1.4426950408889634
S2 = SCALE * LOG2E


def _local_partial(slot, q_buf, o_ref, ml_ref, k_ref, v_ref, ones_ref):
    for h in range(HQ):
        q = q_buf[slot, :, h * DH:(h + 1) * DH]
        k = k_ref[:, h * DH:(h + 1) * DH]
        v = v_ref[:, h * DH:(h + 1) * DH]
        s = lax.dot_general(
            q, k, (((1,), (1,)), ((), ())),
            preferred_element_type=jnp.float32,
        )
        m = jnp.max(s, axis=1, keepdims=True)
        p = jnp.exp2(s - m).astype(jnp.bfloat16)
        pv = lax.dot_general(
            p, v, (((1,), (0,)), ((), ())),
            preferred_element_type=jnp.float32,
        )
        l = lax.dot_general(
            p, ones_ref[...], (((1,), (0,)), ((), ())),
            preferred_element_type=jnp.float32,
        )
        o_ref[:, h * DH:(h + 1) * DH] = pv.astype(jnp.bfloat16)
        ml_ref[0, :, h:h + 1] = m
        ml_ref[1, :, h:h + 1] = l[:, 0:1]


def _body(x_ref, wq_ref, wo_ref, k_ref, v_ref, out_ref,
          q_buf, psend, collect, ml_send, ml_coll, onorm, ones_buf,
          q_ssem, q_rsem, o_ssem, o_rsem, ml_ssem, ml_rsem):
    my = lax.axis_index("i")
    left = lax.rem(my + N_DEV - 1, N_DEV)
    right = lax.rem(my + 1, N_DEV)
    diag = lax.rem(my + 2, N_DEV)

    def q_copy(hop):
        return pltpu.make_async_remote_copy(
            src_ref=q_buf.at[hop], dst_ref=q_buf.at[hop + 1],
            send_sem=q_ssem.at[hop], recv_sem=q_rsem.at[hop],
            device_id=(right,), device_id_type=pl.DeviceIdType.MESH,
        )

    def partial_copy(hop, target):
        ro = pltpu.make_async_remote_copy(
            src_ref=psend.at[hop], dst_ref=collect.at[hop],
            send_sem=o_ssem.at[hop], recv_sem=o_rsem.at[hop],
            device_id=(target,), device_id_type=pl.DeviceIdType.MESH,
        )
        rml = pltpu.make_async_remote_copy(
            src_ref=ml_send.at[hop], dst_ref=ml_coll.at[hop],
            send_sem=ml_ssem.at[hop], recv_sem=ml_rsem.at[hop],
            device_id=(target,), device_id_type=pl.DeviceIdType.MESH,
        )
        return ro, rml

    barrier = pltpu.get_barrier_semaphore()
    for nbr in (left, right, diag):
        pl.semaphore_signal(
            barrier, inc=1, device_id=(nbr,),
            device_id_type=pl.DeviceIdType.MESH,
        )
    pl.semaphore_wait(barrier, 3)

    pending_sends = []
    ones_buf[...] = jnp.ones((SKV, 128), jnp.bfloat16)

    q_buf[0] = (jnp.dot(x_ref[...], wq_ref[...],
                        preferred_element_type=jnp.float32)
                * S2).astype(jnp.bfloat16)
    rq0 = q_copy(0)
    rq0.start()
    pending_sends.append(rq0)

    for hop in range(N_DEV):
        if hop > 0:
            q_copy(hop - 1).wait_recv()
            if hop < N_DEV - 1:
                rq = q_copy(hop)
                rq.start()
                pending_sends.append(rq)

        if hop == 0:
            _local_partial(0, q_buf, collect.at[0], ml_coll.at[0],
                           k_ref, v_ref, ones_buf)
        else:
            _local_partial(hop, q_buf, psend.at[hop], ml_send.at[hop],
                           k_ref, v_ref, ones_buf)
            home = lax.rem(my + N_DEV - hop, N_DEV)
            ro, rml = partial_copy(hop, home)
            ro.start()
            rml.start()
            pending_sends.append(ro)
            pending_sends.append(rml)

    for j in range(1, N_DEV):
        ro_in, rml_in = partial_copy(j, my)
        ro_in.wait_recv()
        rml_in.wait_recv()
        m0 = ml_coll[0, 0]
        l0 = ml_coll[0, 1]
        mj = ml_coll[j, 0]
        lj = ml_coll[j, 1]
        m_new = jnp.maximum(m0, mj)
        a0 = jnp.exp2(m0 - m_new)
        aj = jnp.exp2(mj - m_new)
        ml_coll[0, 0] = m_new
        ml_coll[0, 1] = l0 * a0 + lj * aj
        for h in range(HQ):
            collect[0, :, h * DH:(h + 1) * DH] = (
                collect[0, :, h * DH:(h + 1) * DH] * a0[:, h:h + 1]
                + collect[j, :, h * DH:(h + 1) * DH] * aj[:, h:h + 1]
            ).astype(jnp.bfloat16)

    for h in range(HQ):
        onorm[:, h * DH:(h + 1) * DH] = (
            collect[0, :, h * DH:(h + 1) * DH] / ml_coll[0, 1, :, h:h + 1]
        ).astype(jnp.bfloat16)
    out_ref[...] = jnp.dot(onorm[...], wo_ref[...],
                           preferred_element_type=jnp.float32)

    for r in pending_sends:
        r.wait_send()


def kernel(x, Wq, Wo, K_ext, V_ext):
    xs = x[0].astype(jnp.bfloat16)
    Wq16 = Wq.astype(jnp.bfloat16)
    Wo16 = Wo.astype(jnp.bfloat16)
    K = K_ext[0].reshape(SKV, DM).astype(jnp.bfloat16)
    V = V_ext[0].reshape(SKV, DM).astype(jnp.bfloat16)

    out = pl.pallas_call(
        _body,
        out_shape=jax.ShapeDtypeStruct((SQ, DM), jnp.float32),
        in_specs=[pl.BlockSpec(memory_space=pltpu.VMEM)] * 5,
        out_specs=pl.BlockSpec(memory_space=pltpu.VMEM),
        scratch_shapes=[
            pltpu.VMEM((N_DEV, SQ, DM), jnp.bfloat16),
            pltpu.VMEM((N_DEV, SQ, DM), jnp.bfloat16),
            pltpu.VMEM((N_DEV, SQ, DM), jnp.bfloat16),
            pltpu.VMEM((N_DEV, 2, SQ, HQ), jnp.float32),
            pltpu.VMEM((N_DEV, 2, SQ, HQ), jnp.float32),
            pltpu.VMEM((SQ, DM), jnp.bfloat16),
            pltpu.VMEM((SKV, 128), jnp.bfloat16),
            pltpu.SemaphoreType.DMA((N_DEV,)),
            pltpu.SemaphoreType.DMA((N_DEV,)),
            pltpu.SemaphoreType.DMA((N_DEV,)),
            pltpu.SemaphoreType.DMA((N_DEV,)),
            pltpu.SemaphoreType.DMA((N_DEV,)),
            pltpu.SemaphoreType.DMA((N_DEV,)),
        ],
        compiler_params=pltpu.CompilerParams(
            collective_id=0,
            vmem_limit_bytes=62 * 1024 * 1024,
        ),
    )(xs, Wq16, Wo16, K, V)
    return out[None]


# device time: 108374 ns/iter; 1.0626x vs baseline; 1.0626x over previous
import jax
import jax.numpy as jnp
from jax import lax
from jax.experimental import pallas as pl
from jax.experimental.pallas import tpu as pltpu

N_DEV = 4
SQ = 512
SKV = 2048
HQ = 8
DH = 128
DM = HQ * DH
SCALE = 0.08838834764831843
LOG2E = 1.4426950408889634
S2 = SCALE * LOG2E


def _local_partial(slot, q_buf, o_ref, ml_ref, k_ref, v_ref):
    for h in range(HQ):
        q = q_buf[slot, :, h * DH:(h + 1) * DH]
        k = k_ref[:, h * DH:(h + 1) * DH]
        v = v_ref[:, h * DH:(h + 1) * DH]
        s = lax.dot_general(
            q, k, (((1,), (1,)), ((), ())),
            preferred_element_type=jnp.float32,
        )
        m = jnp.max(s, axis=1, keepdims=True)
        p = jnp.exp2(s - m)
        pv = lax.dot_general(
            p.astype(jnp.bfloat16), v, (((1,), (0,)), ((), ())),
            preferred_element_type=jnp.float32,
        )
        o_ref[:, h * DH:(h + 1) * DH] = pv.astype(jnp.bfloat16)
        ml_ref[0, :, h:h + 1] = m
        ml_ref[1, :, h:h + 1] = jnp.sum(p, axis=1, keepdims=True)


def _body(x_ref, wq_ref, wo_ref, k_ref, v_ref, out_ref,
          q_buf, psend, collect, ml_send, ml_coll, onorm,
          q_ssem, q_rsem, o_ssem, o_rsem, ml_ssem, ml_rsem):
    my = lax.axis_index("i")
    left = lax.rem(my + N_DEV - 1, N_DEV)
    right = lax.rem(my + 1, N_DEV)
    diag = lax.rem(my + 2, N_DEV)

    def q_copy(hop):
        return pltpu.make_async_remote_copy(
            src_ref=q_buf.at[hop], dst_ref=q_buf.at[hop + 1],
            send_sem=q_ssem.at[hop], recv_sem=q_rsem.at[hop],
            device_id=(right,), device_id_type=pl.DeviceIdType.MESH,
        )

    def partial_copy(hop, target):
        ro = pltpu.make_async_remote_copy(
            src_ref=psend.at[hop], dst_ref=collect.at[hop],
            send_sem=o_ssem.at[hop], recv_sem=o_rsem.at[hop],
            device_id=(target,), device_id_type=pl.DeviceIdType.MESH,
        )
        rml = pltpu.make_async_remote_copy(
            src_ref=ml_send.at[hop], dst_ref=ml_coll.at[hop],
            send_sem=ml_ssem.at[hop], recv_sem=ml_rsem.at[hop],
            device_id=(target,), device_id_type=pl.DeviceIdType.MESH,
        )
        return ro, rml

    barrier = pltpu.get_barrier_semaphore()
    for nbr in (left, right, diag):
        pl.semaphore_signal(
            barrier, inc=1, device_id=(nbr,),
            device_id_type=pl.DeviceIdType.MESH,
        )
    pl.semaphore_wait(barrier, 3)

    pending_sends = []

    q_buf[0] = (jnp.dot(x_ref[...], wq_ref[...],
                        preferred_element_type=jnp.float32)
                * S2).astype(jnp.bfloat16)
    rq0 = q_copy(0)
    rq0.start()
    pending_sends.append(rq0)

    for hop in range(N_DEV):
        if hop > 0:
            q_copy(hop - 1).wait_recv()
            if hop < N_DEV - 1:
                rq = q_copy(hop)
                rq.start()
                pending_sends.append(rq)

        if hop == 0:
            _local_partial(0, q_buf, collect.at[0], ml_coll.at[0],
                           k_ref, v_ref)
        else:
            _local_partial(hop, q_buf, psend.at[hop], ml_send.at[hop],
                           k_ref, v_ref)
            home = lax.rem(my + N_DEV - hop, N_DEV)
            ro, rml = partial_copy(hop, home)
            ro.start()
            rml.start()
            pending_sends.append(ro)
            pending_sends.append(rml)

    for j in range(1, N_DEV):
        ro_in, rml_in = partial_copy(j, my)
        ro_in.wait_recv()
        rml_in.wait_recv()
        m0 = ml_coll[0, 0]
        l0 = ml_coll[0, 1]
        mj = ml_coll[j, 0]
        lj = ml_coll[j, 1]
        m_new = jnp.maximum(m0, mj)
        a0 = jnp.exp2(m0 - m_new)
        aj = jnp.exp2(mj - m_new)
        ml_coll[0, 0] = m_new
        ml_coll[0, 1] = l0 * a0 + lj * aj
        for h in range(HQ):
            collect[0, :, h * DH:(h + 1) * DH] = (
                collect[0, :, h * DH:(h + 1) * DH] * a0[:, h:h + 1]
                + collect[j, :, h * DH:(h + 1) * DH] * aj[:, h:h + 1]
            ).astype(jnp.bfloat16)

    for h in range(HQ):
        onorm[:, h * DH:(h + 1) * DH] = (
            collect[0, :, h * DH:(h + 1) * DH] / ml_coll[0, 1, :, h:h + 1]
        ).astype(jnp.bfloat16)
    out_ref[...] = jnp.dot(onorm[...], wo_ref[...],
                           preferred_element_type=jnp.float32)

    for r in pending_sends:
        r.wait_send()


def kernel(x, Wq, Wo, K_ext, V_ext):
    xs = x[0].astype(jnp.bfloat16)
    Wq16 = Wq.astype(jnp.bfloat16)
    Wo16 = Wo.astype(jnp.bfloat16)
    K = K_ext[0].reshape(SKV, DM).astype(jnp.bfloat16)
    V = V_ext[0].reshape(SKV, DM).astype(jnp.bfloat16)

    out = pl.pallas_call(
        _body,
        out_shape=jax.ShapeDtypeStruct((SQ, DM), jnp.float32),
        in_specs=[pl.BlockSpec(memory_space=pltpu.VMEM)] * 5,
        out_specs=pl.BlockSpec(memory_space=pltpu.VMEM),
        scratch_shapes=[
            pltpu.VMEM((N_DEV, SQ, DM), jnp.bfloat16),
            pltpu.VMEM((N_DEV, SQ, DM), jnp.bfloat16),
            pltpu.VMEM((N_DEV, SQ, DM), jnp.bfloat16),
            pltpu.VMEM((N_DEV, 2, SQ, HQ), jnp.float32),
            pltpu.VMEM((N_DEV, 2, SQ, HQ), jnp.float32),
            pltpu.VMEM((SQ, DM), jnp.bfloat16),
            pltpu.SemaphoreType.DMA((N_DEV,)),
            pltpu.SemaphoreType.DMA((N_DEV,)),
            pltpu.SemaphoreType.DMA((N_DEV,)),
            pltpu.SemaphoreType.DMA((N_DEV,)),
            pltpu.SemaphoreType.DMA((N_DEV,)),
            pltpu.SemaphoreType.DMA((N_DEV,)),
        ],
        compiler_params=pltpu.CompilerParams(
            collective_id=0,
            vmem_limit_bytes=62 * 1024 * 1024,
        ),
    )(xs, Wq16, Wo16, K, V)
    return out[None]


# device time: 103475 ns/iter; 1.1129x vs baseline; 1.0473x over previous
import jax
import jax.numpy as jnp
from jax import lax
from jax.experimental import pallas as pl
from jax.experimental.pallas import tpu as pltpu

N_DEV = 4
SQ = 512
SKV = 2048
HQ = 8
DH = 128
DM = HQ * DH
SCALE = 0.08838834764831843
LOG2E = 1.4426950408889634
S2 = SCALE * LOG2E


def _local_partial(slot, q_buf, o_ref, ml_ref, k_ref, v_ref):
    for h in range(HQ):
        q = q_buf[slot, :, h * DH:(h + 1) * DH]
        k = k_ref[:, h * DH:(h + 1) * DH]
        v = v_ref[:, h * DH:(h + 1) * DH]
        s = lax.dot_general(
            q, k, (((1,), (1,)), ((), ())),
            preferred_element_type=jnp.float32,
        )
        m = jnp.max(s, axis=1, keepdims=True)
        p = jnp.exp2(s - m)
        pv = lax.dot_general(
            p.astype(jnp.bfloat16), v, (((1,), (0,)), ((), ())),
            preferred_element_type=jnp.float32,
        )
        o_ref[:, h * DH:(h + 1) * DH] = pv.astype(jnp.bfloat16)
        ml_ref[0, :, h:h + 1] = m
        ml_ref[1, :, h:h + 1] = jnp.sum(p, axis=1, keepdims=True)


def _body(x_ref, wq_ref, wo_ref, k_ref, v_ref, out_ref,
          q_buf, psend, collect, ml_send, ml_coll, onorm,
          q_ssem, q_rsem, o_ssem, o_rsem, ml_ssem, ml_rsem):
    my = lax.axis_index("i")
    left = lax.rem(my + N_DEV - 1, N_DEV)
    right = lax.rem(my + 1, N_DEV)
    diag = lax.rem(my + 2, N_DEV)

    def q_copy(hop):
        return pltpu.make_async_remote_copy(
            src_ref=q_buf.at[hop], dst_ref=q_buf.at[hop + 1],
            send_sem=q_ssem.at[hop], recv_sem=q_rsem.at[hop],
            device_id=(right,), device_id_type=pl.DeviceIdType.MESH,
        )

    def partial_copy(hop, target):
        ro = pltpu.make_async_remote_copy(
            src_ref=psend.at[hop], dst_ref=collect.at[hop],
            send_sem=o_ssem.at[hop], recv_sem=o_rsem.at[hop],
            device_id=(target,), device_id_type=pl.DeviceIdType.MESH,
        )
        rml = pltpu.make_async_remote_copy(
            src_ref=ml_send.at[hop], dst_ref=ml_coll.at[hop],
            send_sem=ml_ssem.at[hop], recv_sem=ml_rsem.at[hop],
            device_id=(target,), device_id_type=pl.DeviceIdType.MESH,
        )
        return ro, rml

    barrier = pltpu.get_barrier_semaphore()
    for nbr in (left, right, diag):
        pl.semaphore_signal(
            barrier, inc=1, device_id=(nbr,),
            device_id_type=pl.DeviceIdType.MESH,
        )
    pl.semaphore_wait(barrier, 3)

    pending_sends = []

    q_buf[0] = (jnp.dot(x_ref[...].astype(jnp.bfloat16),
                        wq_ref[...].astype(jnp.bfloat16),
                        preferred_element_type=jnp.float32)
                * S2).astype(jnp.bfloat16)
    rq0 = q_copy(0)
    rq0.start()
    pending_sends.append(rq0)

    for hop in range(N_DEV):
        if hop > 0:
            q_copy(hop - 1).wait_recv()
            if hop < N_DEV - 1:
                rq = q_copy(hop)
                rq.start()
                pending_sends.append(rq)

        if hop == 0:
            _local_partial(0, q_buf, collect.at[0], ml_coll.at[0],
                           k_ref, v_ref)
        else:
            _local_partial(hop, q_buf, psend.at[hop], ml_send.at[hop],
                           k_ref, v_ref)
            home = lax.rem(my + N_DEV - hop, N_DEV)
            ro, rml = partial_copy(hop, home)
            ro.start()
            rml.start()
            pending_sends.append(ro)
            pending_sends.append(rml)

    for j in range(1, N_DEV):
        ro_in, rml_in = partial_copy(j, my)
        ro_in.wait_recv()
        rml_in.wait_recv()
        m0 = ml_coll[0, 0]
        l0 = ml_coll[0, 1]
        mj = ml_coll[j, 0]
        lj = ml_coll[j, 1]
        m_new = jnp.maximum(m0, mj)
        a0 = jnp.exp2(m0 - m_new)
        aj = jnp.exp2(mj - m_new)
        ml_coll[0, 0] = m_new
        ml_coll[0, 1] = l0 * a0 + lj * aj
        for h in range(HQ):
            collect[0, :, h * DH:(h + 1) * DH] = (
                collect[0, :, h * DH:(h + 1) * DH] * a0[:, h:h + 1]
                + collect[j, :, h * DH:(h + 1) * DH] * aj[:, h:h + 1]
            ).astype(jnp.bfloat16)

    for h in range(HQ):
        onorm[:, h * DH:(h + 1) * DH] = (
            collect[0, :, h * DH:(h + 1) * DH] / ml_coll[0, 1, :, h:h + 1]
        ).astype(jnp.bfloat16)
    out_ref[...] = jnp.dot(onorm[...], wo_ref[...].astype(jnp.bfloat16),
                           preferred_element_type=jnp.float32)

    for r in pending_sends:
        r.wait_send()


def kernel(x, Wq, Wo, K_ext, V_ext):
    xs = x[0]
    K = K_ext[0].reshape(SKV, DM).astype(jnp.bfloat16)
    V = V_ext[0].reshape(SKV, DM).astype(jnp.bfloat16)

    out = pl.pallas_call(
        _body,
        out_shape=jax.ShapeDtypeStruct((SQ, DM), jnp.float32),
        in_specs=[pl.BlockSpec(memory_space=pltpu.VMEM)] * 5,
        out_specs=pl.BlockSpec(memory_space=pltpu.VMEM),
        scratch_shapes=[
            pltpu.VMEM((N_DEV, SQ, DM), jnp.bfloat16),
            pltpu.VMEM((N_DEV, SQ, DM), jnp.bfloat16),
            pltpu.VMEM((N_DEV, SQ, DM), jnp.bfloat16),
            pltpu.VMEM((N_DEV, 2, SQ, HQ), jnp.float32),
            pltpu.VMEM((N_DEV, 2, SQ, HQ), jnp.float32),
            pltpu.VMEM((SQ, DM), jnp.bfloat16),
            pltpu.SemaphoreType.DMA((N_DEV,)),
            pltpu.SemaphoreType.DMA((N_DEV,)),
            pltpu.SemaphoreType.DMA((N_DEV,)),
            pltpu.SemaphoreType.DMA((N_DEV,)),
            pltpu.SemaphoreType.DMA((N_DEV,)),
            pltpu.SemaphoreType.DMA((N_DEV,)),
        ],
        compiler_params=pltpu.CompilerParams(
            collective_id=0,
            vmem_limit_bytes=62 * 1024 * 1024,
        ),
    )(xs, Wq, Wo, K, V)
    return out[None]
